# baseline (speedup 1.0000x reference)
import numpy as np
import jax
import jax.numpy as jnp
from jax.experimental import pallas as pl
from jax.experimental.pallas import tpu as pltpu

_BB = 64


def _conv1_idx():
    idx = np.full((5, 28, 256), 250, np.int32)
    for kh in range(5):
        for jw in range(28):
            for blk in range(2):
                for j2 in range(12):
                    kw = jw - (2 * j2 + blk)
                    if 0 <= kw < 5:
                        for o in range(10):
                            idx[kh, jw, blk * 128 + j2 * 10 + o] = (kh * 5 + kw) * 10 + o
    return idx


def _conv2_idx():
    idx = np.full((5, 128, 256), 5000, np.int32)
    for kh in range(5):
        for j in range(12):
            for c in range(10):
                row = j * 10 + c
                for blk in range(2):
                    for j4 in range(4):
                        kw = j - (2 * j4 + blk)
                        if 0 <= kw < 5:
                            for oc in range(20):
                                idx[kh, row, blk * 128 + j4 * 20 + oc] = (
                                    kh * 1000 + (kw * 10 + c) * 20 + oc)
    return idx


def _bias_idx(nch, reps):
    idx = np.full((1, 128), nch, np.int32)
    for j in range(reps):
        for o in range(nch):
            idx[0, j * nch + o] = o
    return idx


_IDX_W1 = _conv1_idx()
_IDX_W2 = _conv2_idx()
_IDX_B1 = _bias_idx(10, 12)
_IDX_B2 = _bias_idx(20, 4)


def _fused_kernel(x_ref, w1t_ref, b1_ref, w2t_ref, b2_ref,
                  wfc1_ref, bfc1_ref, wfc2_ref, bfc2_ref, out_ref):
    f32 = jnp.float32
    B = x_ref.shape[0]
    x = x_ref[...]
    w1t = w1t_ref[...]
    w2t = w2t_ref[...]
    wfc1 = wfc1_ref[...]

    acc = jnp.dot(x[:, 0:24, :].reshape(B * 24, 28), w1t[0],
                  preferred_element_type=f32)
    for kh in range(1, 5):
        acc = acc + jnp.dot(x[:, kh:kh + 24, :].reshape(B * 24, 28), w1t[kh],
                            preferred_element_type=f32)
    acc = acc.reshape(B, 24, 256)
    m = jnp.maximum(acc[:, :, :128], acc[:, :, 128:])
    m = m.reshape(B, 12, 2, 128)
    m = jnp.maximum(m[:, :, 0, :], m[:, :, 1, :])
    h1 = jnp.maximum(m + b1_ref[...], 0.0)

    acc2 = jnp.dot(h1[:, 0:8, :].reshape(B * 8, 128), w2t[0],
                   preferred_element_type=f32)
    for kh in range(1, 5):
        acc2 = acc2 + jnp.dot(h1[:, kh:kh + 8, :].reshape(B * 8, 128), w2t[kh],
                              preferred_element_type=f32)
    acc2 = acc2.reshape(B, 8, 256)
    m2 = jnp.maximum(acc2[:, :, :128], acc2[:, :, 128:])
    m2 = m2.reshape(B, 4, 2, 128)
    m2 = jnp.maximum(m2[:, :, 0, :], m2[:, :, 1, :])
    h2 = jnp.maximum(m2 + b2_ref[...], 0.0)

    z1 = bfc1_ref[...] + jnp.dot(h2[:, 0, :], wfc1[0], preferred_element_type=f32)
    for hh in range(1, 4):
        z1 = z1 + jnp.dot(h2[:, hh, :], wfc1[hh], preferred_element_type=f32)
    z1 = jnp.maximum(z1, 0.0)

    z2 = jnp.dot(z1, wfc2_ref[...], preferred_element_type=f32) + bfc2_ref[...]
    mz = jnp.max(z2, axis=-1, keepdims=True)
    e = jnp.exp(z2 - mz)
    out_ref[...] = (z2 - mz) - jnp.log(jnp.sum(e, axis=-1, keepdims=True))


def kernel(w1, b1, w2, b2, wfc1, bfc1, wfc2, bfc2, x):
    n = x.shape[0]
    nc = wfc2.shape[1]
    x = x.astype(jnp.float32).reshape(n, 28, 28)

    w1f = jnp.concatenate([w1.reshape(-1), jnp.zeros((1,), jnp.float32)])
    w1t = w1f[_IDX_W1]
    w2f = jnp.concatenate([w2.reshape(-1), jnp.zeros((1,), jnp.float32)])
    w2t = w2f[_IDX_W2]
    b1f = jnp.concatenate([b1.reshape(-1), jnp.zeros((1,), jnp.float32)])
    b1l = b1f[_IDX_B1]
    b2f = jnp.concatenate([b2.reshape(-1), jnp.zeros((1,), jnp.float32)])
    b2l = b2f[_IDX_B2]
    wfc1p = jnp.pad(wfc1.reshape(4, 80, 50), ((0, 0), (0, 48), (0, 0)))

    n_pad = (-(-n // _BB)) * _BB
    if n_pad != n:
        x = jnp.pad(x, ((0, n_pad - n), (0, 0), (0, 0)))

    out = pl.pallas_call(
        _fused_kernel,
        out_shape=jax.ShapeDtypeStruct((n_pad, nc), jnp.float32),
        grid=(n_pad // _BB,),
        in_specs=[
            pl.BlockSpec((_BB, 28, 28), lambda i: (i, 0, 0)),
            pl.BlockSpec((5, 28, 256), lambda i: (0, 0, 0)),
            pl.BlockSpec((1, 128), lambda i: (0, 0)),
            pl.BlockSpec((5, 128, 256), lambda i: (0, 0, 0)),
            pl.BlockSpec((1, 128), lambda i: (0, 0)),
            pl.BlockSpec((4, 128, 50), lambda i: (0, 0, 0)),
            pl.BlockSpec((1, 50), lambda i: (0, 0)),
            pl.BlockSpec((50, nc), lambda i: (0, 0)),
            pl.BlockSpec((1, nc), lambda i: (0, 0)),
        ],
        out_specs=pl.BlockSpec((_BB, nc), lambda i: (i, 0)),
        compiler_params=pltpu.CompilerParams(
            dimension_semantics=("parallel",),
            vmem_limit_bytes=64 * 1024 * 1024,
        ),
    )(x, w1t, b1l, w2t, b2l, wfc1p, bfc1, wfc2, bfc2)

    return out[:n]


# speedup vs baseline: 5.2883x; 5.2883x over previous
import numpy as np
import jax
import jax.numpy as jnp
from jax.experimental import pallas as pl
from jax.experimental.pallas import tpu as pltpu

_BB = 256


def _band_selector(n_in, n_out):
    sel = np.zeros((n_in * n_out, 5), np.float32)
    for i in range(n_in):
        for j in range(n_out):
            kw = i - j
            if 0 <= kw < 5:
                sel[i * n_out + j, kw] = 1.0
    return sel


_SEL1 = _band_selector(28, 24)
_SEL2 = _band_selector(12, 8)


def _pack_w1t(w1):
    r1 = w1.reshape(5, 5, 10).transpose(1, 0, 2).reshape(5, 50)
    t = jnp.dot(_SEL1, r1).reshape(28, 24, 5, 10).transpose(2, 0, 1, 3)
    t = t.reshape(5, 28, 12, 2, 10)
    even = jnp.pad(t[:, :, :, 0, :].reshape(5, 28, 120), ((0, 0),) * 2 + ((0, 8),))
    odd = jnp.pad(t[:, :, :, 1, :].reshape(5, 28, 120), ((0, 0),) * 2 + ((0, 8),))
    return jnp.concatenate([even, odd], axis=-1)


def _pack_w2t(w2):
    r2 = w2.reshape(5, 5, 10, 20).transpose(1, 0, 2, 3).reshape(5, 1000)
    t = jnp.dot(_SEL2, r2).reshape(12, 8, 5, 10, 20).transpose(2, 0, 3, 1, 4)
    t = t.reshape(5, 120, 4, 2, 20)
    pad = ((0, 0), (0, 8), (0, 48))
    even = jnp.pad(t[:, :, :, 0, :].reshape(5, 120, 80), pad)
    odd = jnp.pad(t[:, :, :, 1, :].reshape(5, 120, 80), pad)
    return jnp.concatenate([even, odd], axis=-1)


def _fused_kernel(x_ref, w1t_ref, b1_ref, w2t_ref, b2_ref,
                  wfc1_ref, bfc1_ref, wfc2_ref, bfc2_ref, out_ref):
    f32 = jnp.float32
    B = x_ref.shape[0]
    x = x_ref[...]
    w1t = w1t_ref[...]
    w2t = w2t_ref[...]
    wfc1 = wfc1_ref[...]

    acc = jnp.dot(x[:, 0:24, :].reshape(B * 24, 28), w1t[0],
                  preferred_element_type=f32)
    for kh in range(1, 5):
        acc = acc + jnp.dot(x[:, kh:kh + 24, :].reshape(B * 24, 28), w1t[kh],
                            preferred_element_type=f32)
    acc = acc.reshape(B, 24, 256)
    m = jnp.maximum(acc[:, :, :128], acc[:, :, 128:])
    m = m.reshape(B, 12, 2, 128)
    m = jnp.maximum(m[:, :, 0, :], m[:, :, 1, :])
    h1 = jnp.maximum(m + b1_ref[...], 0.0)

    acc2 = jnp.dot(h1[:, 0:8, :].reshape(B * 8, 128), w2t[0],
                   preferred_element_type=f32)
    for kh in range(1, 5):
        acc2 = acc2 + jnp.dot(h1[:, kh:kh + 8, :].reshape(B * 8, 128), w2t[kh],
                              preferred_element_type=f32)
    acc2 = acc2.reshape(B, 8, 256)
    m2 = jnp.maximum(acc2[:, :, :128], acc2[:, :, 128:])
    m2 = m2.reshape(B, 4, 2, 128)
    m2 = jnp.maximum(m2[:, :, 0, :], m2[:, :, 1, :])
    h2 = jnp.maximum(m2 + b2_ref[...], 0.0)

    z1 = bfc1_ref[...] + jnp.dot(h2[:, 0, :], wfc1[0], preferred_element_type=f32)
    for hh in range(1, 4):
        z1 = z1 + jnp.dot(h2[:, hh, :], wfc1[hh], preferred_element_type=f32)
    z1 = jnp.maximum(z1, 0.0)

    z2 = jnp.dot(z1, wfc2_ref[...], preferred_element_type=f32) + bfc2_ref[...]
    mz = jnp.max(z2, axis=-1, keepdims=True)
    e = jnp.exp(z2 - mz)
    out_ref[...] = (z2 - mz) - jnp.log(jnp.sum(e, axis=-1, keepdims=True))


def kernel(w1, b1, w2, b2, wfc1, bfc1, wfc2, bfc2, x):
    n = x.shape[0]
    nc = wfc2.shape[1]
    x = x.astype(jnp.float32).reshape(n, 28, 28)

    w1t = _pack_w1t(w1)
    w2t = _pack_w2t(w2)
    b1l = jnp.pad(jnp.tile(b1, (1, 12)), ((0, 0), (0, 8)))
    b2l = jnp.pad(jnp.tile(b2, (1, 4)), ((0, 0), (0, 48)))
    wfc1p = jnp.pad(wfc1.reshape(4, 80, 50), ((0, 0), (0, 48), (0, 0)))

    n_pad = (-(-n // _BB)) * _BB
    if n_pad != n:
        x = jnp.pad(x, ((0, n_pad - n), (0, 0), (0, 0)))

    out = pl.pallas_call(
        _fused_kernel,
        out_shape=jax.ShapeDtypeStruct((n_pad, nc), jnp.float32),
        grid=(n_pad // _BB,),
        in_specs=[
            pl.BlockSpec((_BB, 28, 28), lambda i: (i, 0, 0)),
            pl.BlockSpec((5, 28, 256), lambda i: (0, 0, 0)),
            pl.BlockSpec((1, 128), lambda i: (0, 0)),
            pl.BlockSpec((5, 128, 256), lambda i: (0, 0, 0)),
            pl.BlockSpec((1, 128), lambda i: (0, 0)),
            pl.BlockSpec((4, 128, 50), lambda i: (0, 0, 0)),
            pl.BlockSpec((1, 50), lambda i: (0, 0)),
            pl.BlockSpec((50, nc), lambda i: (0, 0)),
            pl.BlockSpec((1, nc), lambda i: (0, 0)),
        ],
        out_specs=pl.BlockSpec((_BB, nc), lambda i: (i, 0)),
        compiler_params=pltpu.CompilerParams(
            dimension_semantics=("parallel",),
            vmem_limit_bytes=64 * 1024 * 1024,
        ),
    )(x, w1t, b1l, w2t, b2l, wfc1p, bfc1, wfc2, bfc2)

    return out[:n]


# speedup vs baseline: 5.4324x; 1.0273x over previous
import numpy as np
import jax
import jax.numpy as jnp
from jax.experimental import pallas as pl
from jax.experimental.pallas import tpu as pltpu

_BB = 512


def _band_selector(n_in, n_out):
    sel = np.zeros((n_in * n_out, 5), np.float32)
    for i in range(n_in):
        for j in range(n_out):
            kw = i - j
            if 0 <= kw < 5:
                sel[i * n_out + j, kw] = 1.0
    return sel


_SEL1 = _band_selector(28, 24)
_SEL2 = _band_selector(12, 8)


def _pack_w1t(w1):
    r1 = w1.reshape(5, 5, 10).transpose(1, 0, 2).reshape(5, 50)
    t = jnp.dot(_SEL1, r1).reshape(28, 24, 5, 10).transpose(2, 0, 1, 3)
    t = t.reshape(5, 28, 12, 2, 10)
    even = jnp.pad(t[:, :, :, 0, :].reshape(5, 28, 120), ((0, 0),) * 2 + ((0, 8),))
    odd = jnp.pad(t[:, :, :, 1, :].reshape(5, 28, 120), ((0, 0),) * 2 + ((0, 8),))
    return jnp.concatenate([even, odd], axis=-1)


def _pack_w2t(w2):
    r2 = w2.reshape(5, 5, 10, 20).transpose(1, 0, 2, 3).reshape(5, 1000)
    t = jnp.dot(_SEL2, r2).reshape(12, 8, 5, 10, 20).transpose(2, 0, 3, 1, 4)
    t = t.reshape(5, 120, 4, 2, 20)
    pad = ((0, 0), (0, 8), (0, 48))
    even = jnp.pad(t[:, :, :, 0, :].reshape(5, 120, 80), pad)
    odd = jnp.pad(t[:, :, :, 1, :].reshape(5, 120, 80), pad)
    return jnp.concatenate([even, odd], axis=-1)


def _fused_kernel(x_ref, w1t_ref, b1_ref, w2t_ref, b2_ref,
                  wfc1_ref, bfc1_ref, wfc2_ref, bfc2_ref, out_ref):
    f32 = jnp.float32
    B = x_ref.shape[0]
    x = x_ref[...]
    w1t = w1t_ref[...]
    w2t = w2t_ref[...]
    wfc1 = wfc1_ref[...]

    acc = jnp.dot(x[:, 0:24, :].reshape(B * 24, 28), w1t[0],
                  preferred_element_type=f32)
    for kh in range(1, 5):
        acc = acc + jnp.dot(x[:, kh:kh + 24, :].reshape(B * 24, 28), w1t[kh],
                            preferred_element_type=f32)
    acc = acc.reshape(B, 24, 256)
    m = jnp.maximum(acc[:, :, :128], acc[:, :, 128:])
    m = m.reshape(B, 12, 2, 128)
    m = jnp.maximum(m[:, :, 0, :], m[:, :, 1, :])
    h1 = jnp.maximum(m + b1_ref[...], 0.0)

    acc2 = jnp.dot(h1[:, 0:8, :].reshape(B * 8, 128), w2t[0],
                   preferred_element_type=f32)
    for kh in range(1, 5):
        acc2 = acc2 + jnp.dot(h1[:, kh:kh + 8, :].reshape(B * 8, 128), w2t[kh],
                              preferred_element_type=f32)
    acc2 = acc2.reshape(B, 8, 256)
    m2 = jnp.maximum(acc2[:, :, :128], acc2[:, :, 128:])
    m2 = m2.reshape(B, 4, 2, 128)
    m2 = jnp.maximum(m2[:, :, 0, :], m2[:, :, 1, :])
    h2 = jnp.maximum(m2 + b2_ref[...], 0.0)

    z1 = bfc1_ref[...] + jnp.dot(h2[:, 0, :], wfc1[0], preferred_element_type=f32)
    for hh in range(1, 4):
        z1 = z1 + jnp.dot(h2[:, hh, :], wfc1[hh], preferred_element_type=f32)
    z1 = jnp.maximum(z1, 0.0)

    z2 = jnp.dot(z1, wfc2_ref[...], preferred_element_type=f32) + bfc2_ref[...]
    mz = jnp.max(z2, axis=-1, keepdims=True)
    e = jnp.exp(z2 - mz)
    out_ref[...] = (z2 - mz) - jnp.log(jnp.sum(e, axis=-1, keepdims=True))


def kernel(w1, b1, w2, b2, wfc1, bfc1, wfc2, bfc2, x):
    n = x.shape[0]
    nc = wfc2.shape[1]
    x = x.astype(jnp.float32).reshape(n, 28, 28)

    w1t = _pack_w1t(w1)
    w2t = _pack_w2t(w2)
    b1l = jnp.pad(jnp.tile(b1, (1, 12)), ((0, 0), (0, 8)))
    b2l = jnp.pad(jnp.tile(b2, (1, 4)), ((0, 0), (0, 48)))
    wfc1p = jnp.pad(wfc1.reshape(4, 80, 50), ((0, 0), (0, 48), (0, 0)))

    n_pad = (-(-n // _BB)) * _BB
    if n_pad != n:
        x = jnp.pad(x, ((0, n_pad - n), (0, 0), (0, 0)))

    out = pl.pallas_call(
        _fused_kernel,
        out_shape=jax.ShapeDtypeStruct((n_pad, nc), jnp.float32),
        grid=(n_pad // _BB,),
        in_specs=[
            pl.BlockSpec((_BB, 28, 28), lambda i: (i, 0, 0)),
            pl.BlockSpec((5, 28, 256), lambda i: (0, 0, 0)),
            pl.BlockSpec((1, 128), lambda i: (0, 0)),
            pl.BlockSpec((5, 128, 256), lambda i: (0, 0, 0)),
            pl.BlockSpec((1, 128), lambda i: (0, 0)),
            pl.BlockSpec((4, 128, 50), lambda i: (0, 0, 0)),
            pl.BlockSpec((1, 50), lambda i: (0, 0)),
            pl.BlockSpec((50, nc), lambda i: (0, 0)),
            pl.BlockSpec((1, nc), lambda i: (0, 0)),
        ],
        out_specs=pl.BlockSpec((_BB, nc), lambda i: (i, 0)),
        compiler_params=pltpu.CompilerParams(
            dimension_semantics=("parallel",),
            vmem_limit_bytes=64 * 1024 * 1024,
        ),
    )(x, w1t, b1l, w2t, b2l, wfc1p, bfc1, wfc2, bfc2)

    return out[:n]
